# baseline (speedup 1.0000x reference)
import functools

import jax
import jax.numpy as jnp
from jax import lax
from jax.experimental import pallas as pl
from jax.experimental.pallas import tpu as pltpu
from jax.experimental.pallas import tpu_sc as plsc

N = 10000
E = 320000
K = 5
NPAD = 10240
TSPAN = 640
W = 64
EP = 327680
EW = EP // W
CH = 32
NCH = EW // 16 // CH
RB = 1000
HWMAX = 64

_MESH = dict(core_axis_name="c", subcore_axis_name="s", num_cores=2,
             num_subcores=16)


def _sp(z):
    return jnp.maximum(z, 0.0) + jnp.log1p(jnp.exp(-jnp.abs(z)))


def _zfill(buf, nrows, ncol16):
    zv = jnp.zeros((16,), jnp.float32)

    def zb(r, carry):
        for cc in range(ncol16):
            buf[r, pl.ds(cc * 16, 16)] = zv
        return carry

    lax.fori_loop(0, nrows, zb, 0)


def _zero_acc(tid, acc, zbuf, och):
    def zb(kq, carry):
        pltpu.sync_copy(zbuf, acc.at[pl.ds(tid * TSPAN + kq * och, och)])
        return carry

    lax.fori_loop(0, TSPAN // och, zb, 0)


def _prop_phase(refs, hw, n_blk, src, dst, a, u=None, b=0.0, v=None, c=0.0,
                och=64, stage=True):
    (cid, tid, row2, col2, norm2, acc, src_sp, e_row, e_col, e_nrm,
     rowbs, tmp_v, u_v, v_v, zbuf, gs, ss) = refs
    hv = hw // 16
    npass = 1 if n_blk == 1 else n_blk // 2

    def scale(rb, wq):
        def grp(g, carry):
            nvals = e_nrm[wq, pl.ds(g * 16, 16)]

            def sub(jj, c2):
                j = g * 16 + jj
                nb = nvals.at[jnp.full((16,), jj, jnp.int32)].get(
                    mode="promise_in_bounds")
                for cc in range(hv):
                    sl = pl.ds(cc * 16, 16)
                    rb[j, sl] = rb[j, sl] * nb
                return c2

            lax.fori_loop(0, 16, sub, 0, unroll=4)
            return carry

        lax.fori_loop(0, W // 16, grp, 0)

    for p in range(npass):
        blk_off = 0 if n_blk == 1 else (cid + 2 * p) * NPAD

        if stage or n_blk > 2:
            def stage_body(kq2, carry):
                k0 = 2 * kq2
                r0 = tid * TSPAN + k0 * och
                r1 = r0 + och
                pltpu.sync_copy(src.at[pl.ds(blk_off + r0, och)], tmp_v)
                pltpu.async_copy(src.at[pl.ds(blk_off + r1, och)], v_v, gs[0])
                pltpu.sync_copy(tmp_v, src_sp.at[pl.ds(r0, och)])
                pltpu.make_async_copy(src.at[pl.ds(blk_off + r1, och)], v_v,
                                      gs[0]).wait()
                pltpu.sync_copy(v_v, src_sp.at[pl.ds(r1, och)])
                return carry

            lax.fori_loop(0, TSPAN // och // 2, stage_body, 0)
            if (TSPAN // och) % 2 == 1:
                rl = tid * TSPAN + (TSPAN // och - 1) * och
                pltpu.sync_copy(src.at[pl.ds(blk_off + rl, och)], tmp_v)
                pltpu.sync_copy(tmp_v, src_sp.at[pl.ds(rl, och)])
            plsc.subcore_barrier()

        def g_issue(wq, rb, sem):
            pltpu.async_copy(src_sp.at[e_row.at[wq]], rb, sem)

        def g_wait(wq, rb, sem):
            pltpu.make_async_copy(src_sp.at[e_row.at[wq]], rb, sem).wait()

        def s_issue(wq, rb, sem):
            pltpu.async_copy(rb, acc.at[e_col.at[wq]], sem, add=True)

        def s_wait(wq, rb, sem):
            pltpu.make_async_copy(rb, acc.at[e_col.at[wq]], sem).wait()

        def chunk_body(cb, carry):
            er0 = tid * (CH * NCH) + cb * CH
            pltpu.sync_copy(row2.at[pl.ds(er0, CH)], e_row)
            pltpu.sync_copy(col2.at[pl.ds(er0, CH)], e_col)
            pltpu.sync_copy(norm2.at[pl.ds(er0, CH)], e_nrm)
            for m in range(3):
                g_issue(m, rowbs[m], gs[m])

            def quad_body(q, carry2):
                for m in range(4):
                    w = 4 * q + m
                    pm = (m - 1) % 4
                    if m == 0:
                        @pl.when(q > 0)
                        def _():
                            s_wait(w - 1, rowbs[pm], ss[pm])

                        g_issue(w + 3, rowbs[pm], gs[pm])
                    else:
                        @pl.when(q < CH // 4 - 1)
                        def _():
                            s_wait(w - 1, rowbs[pm], ss[pm])
                            g_issue(w + 3, rowbs[pm], gs[pm])

                    g_wait(w, rowbs[m], gs[m])
                    scale(rowbs[m], w)
                    s_issue(w, rowbs[m], ss[m])
                return carry2

            lax.fori_loop(0, CH // 4, quad_body, 0)
            for m in range(4):
                s_wait(CH - 4 + m, rowbs[m], ss[m])
            return carry

        lax.fori_loop(0, NCH, chunk_body, 0)
        plsc.subcore_barrier()

        def wout_body(kq, carry):
            r0 = tid * TSPAN + kq * och
            pltpu.sync_copy(acc.at[pl.ds(r0, och)], tmp_v)
            if u is not None:
                pltpu.sync_copy(u.at[pl.ds(blk_off + r0, och)], u_v)
            if v is not None:
                pltpu.sync_copy(v.at[pl.ds(blk_off + r0, och)], v_v)
            pltpu.sync_copy(zbuf, acc.at[pl.ds(r0, och)])

            def comb_body(r, carry2):
                for cc in range(hv):
                    sl = pl.ds(cc * 16, 16)
                    val = tmp_v[r, sl]
                    if a != 1.0:
                        val = val * a
                    if u is not None:
                        uu = u_v[r, sl]
                        val = (val - uu) if b == -1.0 else (val + uu * b)
                    if v is not None:
                        vv = v_v[r, sl]
                        val = (val - vv) if c == -1.0 else (val + vv * c)
                    tmp_v[r, sl] = val
                return carry2

            lax.fori_loop(0, och, comb_body, 0)
            pltpu.sync_copy(tmp_v, dst.at[pl.ds(blk_off + r0, och)])
            if n_blk <= 2:
                pltpu.sync_copy(tmp_v, src_sp.at[pl.ds(r0, och)])
            return carry

        lax.fori_loop(0, TSPAN // och, wout_body, 0)
        plsc.subcore_barrier()


def _sc_scratch(hw, och):
    f32 = jnp.float32
    i32 = jnp.int32
    return [
        pltpu.VMEM_SHARED((NPAD, hw), f32),
        pltpu.VMEM_SHARED((NPAD, hw), f32),
        pltpu.VMEM((CH, W), i32),
        pltpu.VMEM((CH, W), i32),
        pltpu.VMEM((CH, W), f32),
        pltpu.VMEM((W, hw), f32),
        pltpu.VMEM((W, hw), f32),
        pltpu.VMEM((W, hw), f32),
        pltpu.VMEM((W, hw), f32),
        pltpu.VMEM((och, hw), f32),
        pltpu.VMEM((och, hw), f32),
        pltpu.VMEM((och, hw), f32),
        pltpu.VMEM((och, hw), f32),
        pltpu.SemaphoreType.DMA,
        pltpu.SemaphoreType.DMA,
        pltpu.SemaphoreType.DMA,
        pltpu.SemaphoreType.DMA,
        pltpu.SemaphoreType.DMA,
        pltpu.SemaphoreType.DMA,
        pltpu.SemaphoreType.DMA,
        pltpu.SemaphoreType.DMA,
    ]


@functools.lru_cache(maxsize=None)
def _sc_layer_fwd(hw, n_blk):
    hv = hw // 16
    nd = n_blk * NPAD
    och = max(32, min(128, 2048 // hw))
    f32 = jnp.float32

    def body(h2, row2, col2, norm2, t1, t2, t3, t4,
             acc, src_sp, e_row, e_col, e_nrm, r0_, r1_, r2_, r3_,
             tmp_v, u_v, v_v, zbuf, g0, g1, g2, g3, s0, s1, s2, s3):
        cid = lax.axis_index("c")
        tid = lax.axis_index("s")

        def work():
            refs = (cid, tid, row2, col2, norm2, acc, src_sp, e_row, e_col,
                    e_nrm, (r0_, r1_, r2_, r3_),
                    tmp_v, u_v, v_v, zbuf, (g0, g1, g2, g3),
                    (s0, s1, s2, s3))
            _zfill(zbuf, och, hv)
            _zero_acc(tid, acc, zbuf, och)
            plsc.subcore_barrier()
            _prop_phase(refs, hw, n_blk, h2, t1, 1.0, och=och, stage=True)
            _prop_phase(refs, hw, n_blk, t1, t2, 2.0, u=h2, b=-1.0, och=och,
                        stage=False)
            _prop_phase(refs, hw, n_blk, t2, t3, 2.0, u=t1, b=-1.0, och=och,
                        stage=False)
            _prop_phase(refs, hw, n_blk, t3, t4, 2.0, u=t2, b=-1.0, och=och,
                        stage=False)

        if n_blk > 1:
            work()
        else:
            pl.when(cid == 0)(work)

    out = jax.ShapeDtypeStruct((nd, hw), f32)
    return pl.kernel(
        body,
        out_type=[out, out, out, out],
        mesh=plsc.VectorSubcoreMesh(**_MESH),
        compiler_params=pltpu.CompilerParams(use_tc_tiling_on_sc=False),
        scratch_types=_sc_scratch(hw, och),
    )


@functools.lru_cache(maxsize=None)
def _sc_layer0():
    hw = 16
    och = 128
    f32 = jnp.float32

    def body(z0, z1, z2, z3, z4, row2, col2, ew2,
             deginv, norm2, b3, b2, b1, s0_,
             acc, src_sp, e_row, e_col, e_nrm, r0_, r1_, r2_, r3_,
             tmp_v, u_v, v_v, zbuf, g0, g1, g2, g3, s0, s1, s2, s3,
             ew_v, g_v, dacc, dbuf):
        cid = lax.axis_index("c")
        tid = lax.axis_index("s")

        @pl.when(cid == 0)
        def _():
            refs = (cid, tid, row2, col2, norm2, acc, src_sp, e_row, e_col,
                    e_nrm, (r0_, r1_, r2_, r3_),
                    tmp_v, u_v, v_v, zbuf, (g0, g1, g2, g3),
                    (s0, s1, s2, s3))
            _zfill(zbuf, och, 1)
            _zero_acc(tid, acc, zbuf, och)
            zv = jnp.zeros((16,), f32)

            def zd(r, carry):
                dbuf[pl.ds(r * 16, 16)] = zv
                return carry

            lax.fori_loop(0, TSPAN // 16, zd, 0)
            pltpu.sync_copy(dbuf, dacc.at[pl.ds(tid * TSPAN, TSPAN)])
            plsc.subcore_barrier()

            def dchunk(cb, carry):
                er0 = tid * (CH * NCH) + cb * CH
                pltpu.sync_copy(row2.at[pl.ds(er0, CH)], e_row)
                pltpu.sync_copy(ew2.at[pl.ds(er0, CH)], ew_v)

                def dwin(wq, carry2):
                    pltpu.sync_copy(ew_v.at[wq], dacc.at[e_row.at[wq]],
                                    add=True)
                    return carry2

                lax.fori_loop(0, CH, dwin, 0)
                return carry

            lax.fori_loop(0, NCH, dchunk, 0)
            plsc.subcore_barrier()

            pltpu.sync_copy(dacc.at[pl.ds(tid * TSPAN, TSPAN)], dbuf)

            def dinv(r, carry):
                sl = pl.ds(r * 16, 16)
                dd = dbuf[sl]
                dbuf[sl] = jnp.where(dd > 0.0, 1.0 / dd, 0.0)
                return carry

            lax.fori_loop(0, TSPAN // 16, dinv, 0)
            pltpu.sync_copy(dbuf, deginv.at[pl.ds(tid * TSPAN, TSPAN)])
            pltpu.sync_copy(dbuf, dacc.at[pl.ds(tid * TSPAN, TSPAN)])
            plsc.subcore_barrier()

            def nchunk(cb, carry):
                er0 = tid * (CH * NCH) + cb * CH
                pltpu.sync_copy(row2.at[pl.ds(er0, CH)], e_row)
                pltpu.sync_copy(ew2.at[pl.ds(er0, CH)], ew_v)

                def nwin(wq, carry2):
                    pltpu.sync_copy(dacc.at[e_row.at[wq]], g_v)
                    for k8 in range(W // 16):
                        sl = pl.ds(k8 * 16, 16)
                        e_nrm[wq, sl] = -(ew_v[wq, sl] * g_v[sl])
                    return carry2

                lax.fori_loop(0, CH, nwin, 0)
                pltpu.sync_copy(e_nrm, norm2.at[pl.ds(er0, CH)])
                return carry

            lax.fori_loop(0, NCH, nchunk, 0)
            plsc.subcore_barrier()

            _prop_phase(refs, hw, 1, z4, b3, 2.0, u=z3, b=1.0, och=och,
                        stage=True)
            _prop_phase(refs, hw, 1, b3, b2, 2.0, u=z4, b=-1.0, v=z2, c=1.0,
                        och=och, stage=False)
            _prop_phase(refs, hw, 1, b2, b1, 2.0, u=b3, b=-1.0, v=z1, c=1.0,
                        och=och, stage=False)
            _prop_phase(refs, hw, 1, b1, s0_, 1.0, u=b2, b=-1.0, v=z0, c=1.0,
                        och=och, stage=False)

    nodev = jax.ShapeDtypeStruct((NPAD, hw), f32)
    return pl.kernel(
        body,
        out_type=[jax.ShapeDtypeStruct((NPAD,), f32),
                  jax.ShapeDtypeStruct((EW, W), f32),
                  nodev, nodev, nodev, nodev],
        mesh=plsc.VectorSubcoreMesh(**_MESH),
        compiler_params=pltpu.CompilerParams(use_tc_tiling_on_sc=False),
        scratch_types=_sc_scratch(hw, och) + [
            pltpu.VMEM((CH, W), f32),
            pltpu.VMEM((W,), f32),
            pltpu.VMEM_SHARED((NPAD,), f32),
            pltpu.VMEM((TSPAN,), f32),
        ],
    )


def _tc_z(x, w0):
    def bodyz(x_ref, w_ref, z_ref):
        for k in range(K):
            z_ref[k] = jnp.dot(x_ref[...], w_ref[k],
                               preferred_element_type=jnp.float32)

    return pl.pallas_call(
        bodyz,
        grid=(N // RB,),
        in_specs=[pl.BlockSpec((RB, 128), lambda i: (i, 0)),
                  pl.BlockSpec((K, 128, 16), lambda i: (0, 0, 0))],
        out_specs=pl.BlockSpec((K, RB, 16), lambda i: (0, i, 0)),
        out_shape=jax.ShapeDtypeStruct((K, NPAD, 16), jnp.float32),
    )(x, w0)


def _tc_act0(s0, b0):
    def bodya(s_ref, b_ref, o_ref):
        o_ref[...] = _sp(s_ref[...] + b_ref[...])

    return pl.pallas_call(
        bodya,
        grid=(N // RB,),
        in_specs=[pl.BlockSpec((RB, 16), lambda i: (i, 0)),
                  pl.BlockSpec((1, 16), lambda i: (0, 0))],
        out_specs=pl.BlockSpec((RB, 16), lambda i: (i, 0)),
        out_shape=jax.ShapeDtypeStruct((NPAD, 16), jnp.float32),
    )(s0, b0.reshape(1, 16))


def _tc_mm(txs, w_flat, bias, si, hw_in, so, hw_out):
    def bodym(*refs):
        t_refs = refs[:K]
        w_ref, b_ref, o_ref = refs[K], refs[K + 1], refs[K + 2]
        acc = jnp.zeros((RB, hw_out), jnp.float32)
        for k in range(K):
            for s_ in range(si):
                acc = acc + jnp.dot(
                    t_refs[k][s_],
                    w_ref[0, pl.ds((k * si + s_) * hw_in, hw_in), :],
                    preferred_element_type=jnp.float32)
        o_ref[0] = _sp(acc + b_ref[0])

    w_blk = w_flat.reshape(K * si * hw_in, so, hw_out).transpose(1, 0, 2)
    b_blk = bias.reshape(1, so, hw_out).transpose(1, 0, 2)
    tx_spec = pl.BlockSpec((si, RB, hw_in), lambda i, s: (0, i, 0))
    return pl.pallas_call(
        bodym,
        grid=(N // RB, so),
        in_specs=[tx_spec] * K + [
            pl.BlockSpec((1, K * si * hw_in, hw_out), lambda i, s: (s, 0, 0)),
            pl.BlockSpec((1, 1, hw_out), lambda i, s: (s, 0, 0))],
        out_specs=pl.BlockSpec((1, RB, hw_out), lambda i, s: (s, i, 0)),
        out_shape=jax.ShapeDtypeStruct((so, NPAD, hw_out), jnp.float32),
    )(*txs, w_blk, b_blk)


def _tc_final(txs, w_flat, bias, fcw_t, fcb, si, hw_in):
    def bodyf(*refs):
        t_refs = refs[:K]
        w_ref, b_ref, fw_ref, fb_ref, o_ref = refs[K:K + 5]
        acc = jnp.zeros((RB, 512), jnp.float32)
        for k in range(K):
            for s_ in range(si):
                acc = acc + jnp.dot(
                    t_refs[k][s_],
                    w_ref[pl.ds((k * si + s_) * hw_in, hw_in), :],
                    preferred_element_type=jnp.float32)
        h = _sp(acc + b_ref[...])
        o_ref[...] = jnp.dot(h, fw_ref[...],
                             preferred_element_type=jnp.float32) + fb_ref[...]

    tx_spec = pl.BlockSpec((si, RB, hw_in), lambda i: (0, i, 0))
    return pl.pallas_call(
        bodyf,
        grid=(N // RB,),
        in_specs=[tx_spec] * K + [
            pl.BlockSpec((K * si * hw_in, 512), lambda i: (0, 0)),
            pl.BlockSpec((1, 512), lambda i: (0, 0)),
            pl.BlockSpec((512, 128), lambda i: (0, 0)),
            pl.BlockSpec((1, 128), lambda i: (0, 0))],
        out_specs=pl.BlockSpec((RB, 128), lambda i: (i, 0)),
        out_shape=jax.ShapeDtypeStruct((N, 128), jnp.float32),
    )(*txs, w_flat, bias.reshape(1, -1), fcw_t, fcb.reshape(1, -1))


def kernel(x, edge_weigth, params, edge_index, batch):
    f32 = jnp.float32
    pad = EP - E
    ar = (jnp.arange(pad, dtype=jnp.int32) * 13) % N
    row2 = jnp.concatenate([edge_index[0], ar]).reshape(EW, W)
    col2 = jnp.concatenate([edge_index[1], ar]).reshape(EW, W)
    ew2 = jnp.concatenate(
        [edge_weigth, jnp.zeros((pad,), jnp.float32)]).reshape(EW, W)

    dims = [(128, 16), (16, 32), (32, 64), (64, 64), (64, 128), (128, 256),
            (256, 512)]

    z = _tc_z(x, params["W0"])
    zs = [z[k] for k in range(K)]
    deginv, norm2, b3, b2, b1, s0 = _sc_layer0()(*zs, row2, col2, ew2)
    h = _tc_act0(s0, params["b0"])

    si, hw_in = 1, 16
    h_flat = h

    for i in range(1, 7):
        w_in, w_out = dims[i]
        assert si * hw_in == w_in
        t1, t2, t3, t4 = _sc_layer_fwd(hw_in, si)(h_flat, row2, col2, norm2)
        txs = [r.reshape(si, NPAD, hw_in) for r in (h_flat, t1, t2, t3, t4)]
        w_flat = params["W%d" % i].reshape(K * w_in, w_out)
        if i < 6:
            hw_out = 16 if w_out == 16 else min(HWMAX, w_out // 2)
            so = w_out // hw_out
            h_nd = _tc_mm(txs, w_flat, params["b%d" % i], si, hw_in, so,
                          hw_out)
            h_flat = h_nd.reshape(so * NPAD, hw_out)
            si, hw_in = so, hw_out
        else:
            fcw_t = jnp.zeros((512, 128), f32).at[:, :3].set(
                params["fc_w"].T.astype(f32))
            fcb = jnp.zeros((128,), f32).at[:3].set(params["fc_b"])
            outp = _tc_final(txs, w_flat, params["b%d" % i], fcw_t, fcb,
                             si, hw_in)
            return outp[:, :3]


# speedup vs baseline: 1.0073x; 1.0073x over previous
import functools

import jax
import jax.numpy as jnp
from jax import lax
from jax.experimental import pallas as pl
from jax.experimental.pallas import tpu as pltpu
from jax.experimental.pallas import tpu_sc as plsc

N = 10000
E = 320000
K = 5
NPAD = 10240
TSPAN = 640
W = 64
EP = 327680
EW = EP // W
CH = 32
NCH = EW // 16 // CH
RB = 1000
HWMAX = 64

_MESH = dict(core_axis_name="c", subcore_axis_name="s", num_cores=2,
             num_subcores=16)


def _sp(z):
    return jnp.maximum(z, 0.0) + jnp.log1p(jnp.exp(-jnp.abs(z)))


def _zfill(buf, nrows, ncol16):
    zv = jnp.zeros((16,), jnp.float32)

    def zb(r, carry):
        for cc in range(ncol16):
            buf[r, pl.ds(cc * 16, 16)] = zv
        return carry

    lax.fori_loop(0, nrows, zb, 0)


def _zero_acc(tid, acc, zbuf, och):
    def zb(kq, carry):
        pltpu.sync_copy(zbuf, acc.at[pl.ds(tid * TSPAN + kq * och, och)])
        return carry

    lax.fori_loop(0, TSPAN // och, zb, 0)


def _prop_phase(refs, hw, n_blk, src, dst, a, u=None, b=0.0, v=None, c=0.0,
                och=64, stage=True):
    (cid, tid, row2, col2, norm2, acc, src_sp, e_row, e_col, e_nrm,
     rowbs, tmp_v, u_v, v_v, zbuf, gs, ss) = refs
    hv = hw // 16
    npass = 1 if n_blk == 1 else n_blk // 2

    def scale(rb, wq):
        def grp(g, carry):
            nvals = e_nrm[wq, pl.ds(g * 16, 16)]

            def sub(jj, c2):
                j = g * 16 + jj
                nb = nvals.at[jnp.full((16,), jj, jnp.int32)].get(
                    mode="promise_in_bounds")
                for cc in range(hv):
                    sl = pl.ds(cc * 16, 16)
                    rb[j, sl] = rb[j, sl] * nb
                return c2

            lax.fori_loop(0, 16, sub, 0, unroll=4)
            return carry

        lax.fori_loop(0, W // 16, grp, 0)

    for p in range(npass):
        blk_off = 0 if n_blk == 1 else (cid + 2 * p) * NPAD

        if stage or n_blk > 2:
            def stage_body(kq2, carry):
                k0 = 2 * kq2
                r0 = tid * TSPAN + k0 * och
                r1 = r0 + och
                pltpu.sync_copy(src.at[pl.ds(blk_off + r0, och)], tmp_v)
                pltpu.async_copy(src.at[pl.ds(blk_off + r1, och)], v_v, gs[0])
                pltpu.sync_copy(tmp_v, src_sp.at[pl.ds(r0, och)])
                pltpu.make_async_copy(src.at[pl.ds(blk_off + r1, och)], v_v,
                                      gs[0]).wait()
                pltpu.sync_copy(v_v, src_sp.at[pl.ds(r1, och)])
                return carry

            lax.fori_loop(0, TSPAN // och // 2, stage_body, 0)
            if (TSPAN // och) % 2 == 1:
                rl = tid * TSPAN + (TSPAN // och - 1) * och
                pltpu.sync_copy(src.at[pl.ds(blk_off + rl, och)], tmp_v)
                pltpu.sync_copy(tmp_v, src_sp.at[pl.ds(rl, och)])
            plsc.subcore_barrier()

        def g_issue(wq, rb, sem):
            pltpu.async_copy(src_sp.at[e_row.at[wq]], rb, sem)

        def g_wait(wq, rb, sem):
            pltpu.make_async_copy(src_sp.at[e_row.at[wq]], rb, sem).wait()

        def s_issue(wq, rb, sem):
            pltpu.async_copy(rb, acc.at[e_col.at[wq]], sem, add=True)

        def s_wait(wq, rb, sem):
            pltpu.make_async_copy(rb, acc.at[e_col.at[wq]], sem).wait()

        def chunk_body(cb, carry):
            er0 = tid * (CH * NCH) + cb * CH
            pltpu.sync_copy(row2.at[pl.ds(er0, CH)], e_row)
            pltpu.sync_copy(col2.at[pl.ds(er0, CH)], e_col)
            pltpu.sync_copy(norm2.at[pl.ds(er0, CH)], e_nrm)
            for m in range(3):
                g_issue(m, rowbs[m], gs[m])

            def quad_body(q, carry2):
                for m in range(4):
                    w = 4 * q + m
                    pm = (m - 1) % 4
                    if m == 0:
                        @pl.when(q > 0)
                        def _():
                            s_wait(w - 1, rowbs[pm], ss[pm])

                        g_issue(w + 3, rowbs[pm], gs[pm])
                    else:
                        @pl.when(q < CH // 4 - 1)
                        def _():
                            s_wait(w - 1, rowbs[pm], ss[pm])
                            g_issue(w + 3, rowbs[pm], gs[pm])

                    g_wait(w, rowbs[m], gs[m])
                    scale(rowbs[m], w)
                    s_issue(w, rowbs[m], ss[m])
                return carry2

            lax.fori_loop(0, CH // 4, quad_body, 0)
            for m in range(4):
                s_wait(CH - 4 + m, rowbs[m], ss[m])
            return carry

        lax.fori_loop(0, NCH, chunk_body, 0)
        plsc.subcore_barrier()

        def wout_body(kq, carry):
            r0 = tid * TSPAN + kq * och
            pltpu.sync_copy(acc.at[pl.ds(r0, och)], tmp_v)
            if u is not None:
                pltpu.sync_copy(u.at[pl.ds(blk_off + r0, och)], u_v)
            if v is not None:
                pltpu.sync_copy(v.at[pl.ds(blk_off + r0, och)], v_v)
            pltpu.sync_copy(zbuf, acc.at[pl.ds(r0, och)])

            def comb_body(r, carry2):
                for cc in range(hv):
                    sl = pl.ds(cc * 16, 16)
                    val = tmp_v[r, sl]
                    if a != 1.0:
                        val = val * a
                    if u is not None:
                        uu = u_v[r, sl]
                        val = (val - uu) if b == -1.0 else (val + uu * b)
                    if v is not None:
                        vv = v_v[r, sl]
                        val = (val - vv) if c == -1.0 else (val + vv * c)
                    tmp_v[r, sl] = val
                return carry2

            lax.fori_loop(0, och, comb_body, 0)
            pltpu.sync_copy(tmp_v, dst.at[pl.ds(blk_off + r0, och)])
            if n_blk <= 2:
                pltpu.sync_copy(tmp_v, src_sp.at[pl.ds(r0, och)])
            return carry

        lax.fori_loop(0, TSPAN // och, wout_body, 0)
        plsc.subcore_barrier()


def _sc_scratch(hw, och):
    f32 = jnp.float32
    i32 = jnp.int32
    return [
        pltpu.VMEM_SHARED((NPAD, hw), f32),
        pltpu.VMEM_SHARED((NPAD, hw), f32),
        pltpu.VMEM((CH, W), i32),
        pltpu.VMEM((CH, W), i32),
        pltpu.VMEM((CH, W), f32),
        pltpu.VMEM((W, hw), f32),
        pltpu.VMEM((W, hw), f32),
        pltpu.VMEM((W, hw), f32),
        pltpu.VMEM((W, hw), f32),
        pltpu.VMEM((och, hw), f32),
        pltpu.VMEM((och, hw), f32),
        pltpu.VMEM((och, hw), f32),
        pltpu.VMEM((och, hw), f32),
        pltpu.SemaphoreType.DMA,
        pltpu.SemaphoreType.DMA,
        pltpu.SemaphoreType.DMA,
        pltpu.SemaphoreType.DMA,
        pltpu.SemaphoreType.DMA,
        pltpu.SemaphoreType.DMA,
        pltpu.SemaphoreType.DMA,
        pltpu.SemaphoreType.DMA,
    ]


@functools.lru_cache(maxsize=None)
def _sc_layer_fwd(hw, n_blk):
    hv = hw // 16
    nd = n_blk * NPAD
    och = max(32, min(128, 2048 // hw))
    f32 = jnp.float32

    def body(h2, row2, col2, norm2, t1, t2, t3, t4,
             acc, src_sp, e_row, e_col, e_nrm, r0_, r1_, r2_, r3_,
             tmp_v, u_v, v_v, zbuf, g0, g1, g2, g3, s0, s1, s2, s3):
        cid = lax.axis_index("c")
        tid = lax.axis_index("s")

        def work():
            refs = (cid, tid, row2, col2, norm2, acc, src_sp, e_row, e_col,
                    e_nrm, (r0_, r1_, r2_, r3_),
                    tmp_v, u_v, v_v, zbuf, (g0, g1, g2, g3),
                    (s0, s1, s2, s3))
            _zfill(zbuf, och, hv)
            _zero_acc(tid, acc, zbuf, och)
            plsc.subcore_barrier()
            _prop_phase(refs, hw, n_blk, h2, t1, 1.0, och=och, stage=True)
            _prop_phase(refs, hw, n_blk, t1, t2, 2.0, u=h2, b=-1.0, och=och,
                        stage=False)
            _prop_phase(refs, hw, n_blk, t2, t3, 2.0, u=t1, b=-1.0, och=och,
                        stage=False)
            _prop_phase(refs, hw, n_blk, t3, t4, 2.0, u=t2, b=-1.0, och=och,
                        stage=False)

        if n_blk > 1:
            work()
        else:
            pl.when(cid == 0)(work)

    out = jax.ShapeDtypeStruct((nd, hw), f32)
    return pl.kernel(
        body,
        out_type=[out, out, out, out],
        mesh=plsc.VectorSubcoreMesh(**_MESH),
        compiler_params=pltpu.CompilerParams(use_tc_tiling_on_sc=False),
        scratch_types=_sc_scratch(hw, och),
    )


@functools.lru_cache(maxsize=None)
def _sc_layer0():
    hw = 16
    och = 128
    f32 = jnp.float32

    def body(z0, z1, z2, z3, z4, row2, col2, ew2,
             deginv, norm2, b3, b2, b1, s0_,
             acc, src_sp, e_row, e_col, e_nrm, r0_, r1_, r2_, r3_,
             tmp_v, u_v, v_v, zbuf, g0, g1, g2, g3, s0, s1, s2, s3,
             ew_v, g_v, g2_v, dacc, dbuf):
        cid = lax.axis_index("c")
        tid = lax.axis_index("s")

        @pl.when(cid == 0)
        def _():
            refs = (cid, tid, row2, col2, norm2, acc, src_sp, e_row, e_col,
                    e_nrm, (r0_, r1_, r2_, r3_),
                    tmp_v, u_v, v_v, zbuf, (g0, g1, g2, g3),
                    (s0, s1, s2, s3))
            _zfill(zbuf, och, 1)
            _zero_acc(tid, acc, zbuf, och)
            zv = jnp.zeros((16,), f32)

            def zd(r, carry):
                dbuf[pl.ds(r * 16, 16)] = zv
                return carry

            lax.fori_loop(0, TSPAN // 16, zd, 0)
            pltpu.sync_copy(dbuf, dacc.at[pl.ds(tid * TSPAN, TSPAN)])
            plsc.subcore_barrier()

            def dchunk(cb, carry):
                er0 = tid * (CH * NCH) + cb * CH
                pltpu.sync_copy(row2.at[pl.ds(er0, CH)], e_row)
                pltpu.sync_copy(ew2.at[pl.ds(er0, CH)], ew_v)

                def dwin(wq, carry2):
                    pltpu.async_copy(ew_v.at[wq], dacc.at[e_row.at[wq]], s0,
                                     add=True)
                    return carry2

                lax.fori_loop(0, CH, dwin, 0)

                def ddrain(wq, carry2):
                    pltpu.make_async_copy(ew_v.at[wq],
                                          dacc.at[e_row.at[wq]], s0).wait()
                    return carry2

                lax.fori_loop(0, CH, ddrain, 0)
                return carry

            lax.fori_loop(0, NCH, dchunk, 0)
            plsc.subcore_barrier()

            pltpu.sync_copy(dacc.at[pl.ds(tid * TSPAN, TSPAN)], dbuf)

            def dinv(r, carry):
                sl = pl.ds(r * 16, 16)
                dd = dbuf[sl]
                dbuf[sl] = jnp.where(dd > 0.0, 1.0 / dd, 0.0)
                return carry

            lax.fori_loop(0, TSPAN // 16, dinv, 0)
            pltpu.sync_copy(dbuf, deginv.at[pl.ds(tid * TSPAN, TSPAN)])
            pltpu.sync_copy(dbuf, dacc.at[pl.ds(tid * TSPAN, TSPAN)])
            plsc.subcore_barrier()

            def nchunk(cb, carry):
                er0 = tid * (CH * NCH) + cb * CH
                pltpu.sync_copy(row2.at[pl.ds(er0, CH)], e_row)
                pltpu.sync_copy(ew2.at[pl.ds(er0, CH)], ew_v)

                def ncomp(wq, gb):
                    for k8 in range(W // 16):
                        sl = pl.ds(k8 * 16, 16)
                        e_nrm[wq, sl] = -(ew_v[wq, sl] * gb[sl])

                pltpu.async_copy(dacc.at[e_row.at[0]], g_v, g0)

                def nwin(wq2, carry2):
                    w0 = 2 * wq2
                    w1 = w0 + 1
                    pltpu.async_copy(dacc.at[e_row.at[w1]], g2_v, g1)
                    pltpu.make_async_copy(dacc.at[e_row.at[w0]], g_v,
                                          g0).wait()
                    ncomp(w0, g_v)

                    @pl.when(wq2 < CH // 2 - 1)
                    def _():
                        pltpu.async_copy(dacc.at[e_row.at[w0 + 2]], g_v, g0)

                    pltpu.make_async_copy(dacc.at[e_row.at[w1]], g2_v,
                                          g1).wait()
                    ncomp(w1, g2_v)
                    return carry2

                lax.fori_loop(0, CH // 2, nwin, 0)
                pltpu.sync_copy(e_nrm, norm2.at[pl.ds(er0, CH)])
                return carry

            lax.fori_loop(0, NCH, nchunk, 0)
            plsc.subcore_barrier()

            _prop_phase(refs, hw, 1, z4, b3, 2.0, u=z3, b=1.0, och=och,
                        stage=True)
            _prop_phase(refs, hw, 1, b3, b2, 2.0, u=z4, b=-1.0, v=z2, c=1.0,
                        och=och, stage=False)
            _prop_phase(refs, hw, 1, b2, b1, 2.0, u=b3, b=-1.0, v=z1, c=1.0,
                        och=och, stage=False)
            _prop_phase(refs, hw, 1, b1, s0_, 1.0, u=b2, b=-1.0, v=z0, c=1.0,
                        och=och, stage=False)

    nodev = jax.ShapeDtypeStruct((NPAD, hw), f32)
    return pl.kernel(
        body,
        out_type=[jax.ShapeDtypeStruct((NPAD,), f32),
                  jax.ShapeDtypeStruct((EW, W), f32),
                  nodev, nodev, nodev, nodev],
        mesh=plsc.VectorSubcoreMesh(**_MESH),
        compiler_params=pltpu.CompilerParams(use_tc_tiling_on_sc=False),
        scratch_types=_sc_scratch(hw, och) + [
            pltpu.VMEM((CH, W), f32),
            pltpu.VMEM((W,), f32),
            pltpu.VMEM((W,), f32),
            pltpu.VMEM_SHARED((NPAD,), f32),
            pltpu.VMEM((TSPAN,), f32),
        ],
    )


def _tc_z(x, w0):
    def bodyz(x_ref, w_ref, z_ref):
        for k in range(K):
            z_ref[k] = jnp.dot(x_ref[...], w_ref[k],
                               preferred_element_type=jnp.float32)

    return pl.pallas_call(
        bodyz,
        grid=(N // RB,),
        in_specs=[pl.BlockSpec((RB, 128), lambda i: (i, 0)),
                  pl.BlockSpec((K, 128, 16), lambda i: (0, 0, 0))],
        out_specs=pl.BlockSpec((K, RB, 16), lambda i: (0, i, 0)),
        out_shape=jax.ShapeDtypeStruct((K, NPAD, 16), jnp.float32),
    )(x, w0)


def _tc_act0(s0, b0):
    def bodya(s_ref, b_ref, o_ref):
        o_ref[...] = _sp(s_ref[...] + b_ref[...])

    return pl.pallas_call(
        bodya,
        grid=(N // RB,),
        in_specs=[pl.BlockSpec((RB, 16), lambda i: (i, 0)),
                  pl.BlockSpec((1, 16), lambda i: (0, 0))],
        out_specs=pl.BlockSpec((RB, 16), lambda i: (i, 0)),
        out_shape=jax.ShapeDtypeStruct((NPAD, 16), jnp.float32),
    )(s0, b0.reshape(1, 16))


def _tc_mm(txs, w_flat, bias, si, hw_in, so, hw_out):
    def bodym(*refs):
        t_refs = refs[:K]
        w_ref, b_ref, o_ref = refs[K], refs[K + 1], refs[K + 2]
        acc = jnp.zeros((RB, hw_out), jnp.float32)
        for k in range(K):
            for s_ in range(si):
                acc = acc + jnp.dot(
                    t_refs[k][s_],
                    w_ref[0, pl.ds((k * si + s_) * hw_in, hw_in), :],
                    preferred_element_type=jnp.float32)
        o_ref[0] = _sp(acc + b_ref[0])

    w_blk = w_flat.reshape(K * si * hw_in, so, hw_out).transpose(1, 0, 2)
    b_blk = bias.reshape(1, so, hw_out).transpose(1, 0, 2)
    tx_spec = pl.BlockSpec((si, RB, hw_in), lambda i, s: (0, i, 0))
    return pl.pallas_call(
        bodym,
        grid=(N // RB, so),
        in_specs=[tx_spec] * K + [
            pl.BlockSpec((1, K * si * hw_in, hw_out), lambda i, s: (s, 0, 0)),
            pl.BlockSpec((1, 1, hw_out), lambda i, s: (s, 0, 0))],
        out_specs=pl.BlockSpec((1, RB, hw_out), lambda i, s: (s, i, 0)),
        out_shape=jax.ShapeDtypeStruct((so, NPAD, hw_out), jnp.float32),
    )(*txs, w_blk, b_blk)


def _tc_final(txs, w_flat, bias, fcw_t, fcb, si, hw_in):
    def bodyf(*refs):
        t_refs = refs[:K]
        w_ref, b_ref, fw_ref, fb_ref, o_ref = refs[K:K + 5]
        acc = jnp.zeros((RB, 512), jnp.float32)
        for k in range(K):
            for s_ in range(si):
                acc = acc + jnp.dot(
                    t_refs[k][s_],
                    w_ref[pl.ds((k * si + s_) * hw_in, hw_in), :],
                    preferred_element_type=jnp.float32)
        h = _sp(acc + b_ref[...])
        o_ref[...] = jnp.dot(h, fw_ref[...],
                             preferred_element_type=jnp.float32) + fb_ref[...]

    tx_spec = pl.BlockSpec((si, RB, hw_in), lambda i: (0, i, 0))
    return pl.pallas_call(
        bodyf,
        grid=(N // RB,),
        in_specs=[tx_spec] * K + [
            pl.BlockSpec((K * si * hw_in, 512), lambda i: (0, 0)),
            pl.BlockSpec((1, 512), lambda i: (0, 0)),
            pl.BlockSpec((512, 128), lambda i: (0, 0)),
            pl.BlockSpec((1, 128), lambda i: (0, 0))],
        out_specs=pl.BlockSpec((RB, 128), lambda i: (i, 0)),
        out_shape=jax.ShapeDtypeStruct((N, 128), jnp.float32),
    )(*txs, w_flat, bias.reshape(1, -1), fcw_t, fcb.reshape(1, -1))


def kernel(x, edge_weigth, params, edge_index, batch):
    f32 = jnp.float32
    pad = EP - E
    ar = (jnp.arange(pad, dtype=jnp.int32) * 13) % N
    row2 = jnp.concatenate([edge_index[0], ar]).reshape(EW, W)
    col2 = jnp.concatenate([edge_index[1], ar]).reshape(EW, W)
    ew2 = jnp.concatenate(
        [edge_weigth, jnp.zeros((pad,), jnp.float32)]).reshape(EW, W)

    dims = [(128, 16), (16, 32), (32, 64), (64, 64), (64, 128), (128, 256),
            (256, 512)]

    z = _tc_z(x, params["W0"])
    zs = [z[k] for k in range(K)]
    deginv, norm2, b3, b2, b1, s0 = _sc_layer0()(*zs, row2, col2, ew2)
    h = _tc_act0(s0, params["b0"])

    si, hw_in = 1, 16
    h_flat = h

    for i in range(1, 7):
        w_in, w_out = dims[i]
        assert si * hw_in == w_in
        t1, t2, t3, t4 = _sc_layer_fwd(hw_in, si)(h_flat, row2, col2, norm2)
        txs = [r.reshape(si, NPAD, hw_in) for r in (h_flat, t1, t2, t3, t4)]
        w_flat = params["W%d" % i].reshape(K * w_in, w_out)
        if i < 6:
            hw_out = 16 if w_out == 16 else min(HWMAX, w_out // 2)
            so = w_out // hw_out
            h_nd = _tc_mm(txs, w_flat, params["b%d" % i], si, hw_in, so,
                          hw_out)
            h_flat = h_nd.reshape(so * NPAD, hw_out)
            si, hw_in = so, hw_out
        else:
            fcw_t = jnp.zeros((512, 128), f32).at[:, :3].set(
                params["fc_w"].T.astype(f32))
            fcb = jnp.zeros((128,), f32).at[:3].set(params["fc_b"])
            outp = _tc_final(txs, w_flat, params["b%d" % i], fcw_t, fcb,
                             si, hw_in)
            return outp[:, :3]
